# baseline (speedup 1.0000x reference)
import functools

import jax
import jax.numpy as jnp
from jax import lax
from jax.experimental import pallas as pl
from jax.experimental.pallas import tpu as pltpu
from jax.experimental.pallas import tpu_sc as plsc

B_BLK = 8
_L = 16


def _vec_sqrt(x):
    y = 0.5 * (x + 1.0)
    for _ in range(14):
        y = 0.5 * (y + x / y)
    return y


def _sc_renorm_row(row_hbm, out_hbm, row_v, out_v, parity):
    d = 128
    nch = d // _L
    pltpu.sync_copy(row_hbm, row_v)
    ss = jnp.zeros((_L,), jnp.float32)
    for c in range(nch):
        v = row_v[pl.ds(c * _L, _L)]
        ss = ss + v * v
    tot_s = ss[0]
    for i in range(1, _L):
        tot_s = tot_s + ss[i]
    tot = jnp.full((_L,), tot_s, jnp.float32)
    norm = _vec_sqrt(tot)
    scale = jnp.where(norm > 1.0, 1.0 / (norm + 1e-7), jnp.float32(1.0))
    zero = jnp.zeros((_L,), jnp.float32)
    for j in range(2 * nch):
        out_v[pl.ds(j * _L, _L)] = zero
    base_iota = lax.iota(jnp.int32, _L)
    for c in range(nch):
        v = row_v[pl.ds(c * _L, _L)] * scale
        idx = 2 * (c * _L + base_iota) + parity
        plsc.store_scatter(out_v, [idx], v)
    pltpu.sync_copy(out_v, out_hbm)


def _sc_lookup(le_hbm, pe_hbm, enc_hbm, row_v, out_v):
    wid = lax.axis_index("s") + lax.axis_index("c") * 0
    ns = le_hbm.shape[0]
    np_ = pe_hbm.shape[0]

    @pl.when(wid < _L)
    def _le_job():
        _sc_renorm_row(le_hbm.at[wid], enc_hbm.at[wid], row_v, out_v, 0)

    @pl.when(wid < ns - _L)
    def _le_job2():
        _sc_renorm_row(le_hbm.at[wid + _L], enc_hbm.at[wid + _L], row_v, out_v, 0)

    @pl.when(wid < np_)
    def _pe_job():
        _sc_renorm_row(pe_hbm.at[wid], enc_hbm.at[ns + wid], row_v, out_v, 1)


def _tc_body(x_ref, enc_ref, o_ref, a_ref):
    @pl.when(pl.program_id(0) == 0)
    def _build_add():
        ns = a_ref.shape[0]
        ale = enc_ref[:ns, :]
        ape = enc_ref[ns:, :]
        a_ref[...] = ale[:, None, :] + ape[None, :, :]

    o_ref[...] = x_ref[...] + a_ref[...][None]


def kernel(x, learned_table, person_table, num_people):
    del num_people
    b, s, p, d = x.shape
    pe_rows = lax.slice(person_table, (0, 0), (p, person_table.shape[1]))

    sc_call = functools.partial(
        pl.kernel,
        mesh=plsc.VectorSubcoreMesh(core_axis_name="c", subcore_axis_name="s", num_cores=1),
        out_type=jax.ShapeDtypeStruct((s + p, d), jnp.float32),
        scratch_types=[
            pltpu.VMEM((d // 2,), jnp.float32),
            pltpu.VMEM((d,), jnp.float32),
        ],
        compiler_params=pltpu.CompilerParams(needs_layout_passes=False),
    )(_sc_lookup)
    enc = sc_call(learned_table, pe_rows)

    grid = (b // B_BLK,)
    return pl.pallas_call(
        _tc_body,
        grid=grid,
        in_specs=[
            pl.BlockSpec((B_BLK, s, p, d), lambda i: (i, 0, 0, 0)),
            pl.BlockSpec((s + p, d), lambda i: (0, 0)),
        ],
        out_specs=pl.BlockSpec((B_BLK, s, p, d), lambda i: (i, 0, 0, 0)),
        out_shape=jax.ShapeDtypeStruct(x.shape, x.dtype),
        scratch_shapes=[pltpu.VMEM((s, p, d), jnp.float32)],
        compiler_params=pltpu.CompilerParams(
            dimension_semantics=("arbitrary",),
        ),
    )(x, enc)


# speedup vs baseline: 1.0620x; 1.0620x over previous
import functools

import jax
import jax.numpy as jnp
from jax import lax
from jax.experimental import pallas as pl
from jax.experimental.pallas import tpu as pltpu
from jax.experimental.pallas import tpu_sc as plsc

B_BLK = 32
_L = 16


def _vec_sqrt(x):
    y = 0.5 * (x + 1.0)
    for _ in range(14):
        y = 0.5 * (y + x / y)
    return y


def _sc_compute_row(row_v, out_v, parity):
    d = 128
    nch = d // _L
    ss = jnp.zeros((_L,), jnp.float32)
    for c in range(nch):
        v = row_v[pl.ds(c * _L, _L)]
        ss = ss + v * v
    tot_s = ss[0]
    for i in range(1, _L):
        tot_s = tot_s + ss[i]
    tot = jnp.full((_L,), tot_s, jnp.float32)
    norm = _vec_sqrt(tot)
    scale = jnp.where(norm > 1.0, 1.0 / (norm + 1e-7), jnp.float32(1.0))
    zero = jnp.zeros((_L,), jnp.float32)
    for j in range(2 * nch):
        out_v[pl.ds(j * _L, _L)] = zero
    base_iota = lax.iota(jnp.int32, _L)
    for c in range(nch):
        v = row_v[pl.ds(c * _L, _L)] * scale
        idx = 2 * (c * _L + base_iota) + parity
        plsc.store_scatter(out_v, [idx], v)


def _sc_lookup(le_hbm, pe_hbm, enc_hbm, row0_v, row1_v, row2_v, out0_v, out1_v, out2_v, sem0, sem1, sem2):
    wid = lax.axis_index("s") + lax.axis_index("c") * 0
    ns = le_hbm.shape[0]

    cp0 = pltpu.make_async_copy(le_hbm.at[wid], row0_v, sem0)
    cp2 = pltpu.make_async_copy(pe_hbm.at[wid], row2_v, sem2)
    cp0.start()
    cp2.start()

    @pl.when(wid < ns - _L)
    def _start1():
        pltpu.make_async_copy(le_hbm.at[wid + _L], row1_v, sem1).start()

    cp0.wait()
    _sc_compute_row(row0_v, out0_v, 0)
    o0 = pltpu.make_async_copy(out0_v, enc_hbm.at[wid], sem0)
    o0.start()

    cp2.wait()
    _sc_compute_row(row2_v, out2_v, 1)
    o2 = pltpu.make_async_copy(out2_v, enc_hbm.at[ns + wid], sem2)
    o2.start()

    @pl.when(wid < ns - _L)
    def _job1():
        pltpu.make_async_copy(le_hbm.at[wid + _L], row1_v, sem1).wait()
        _sc_compute_row(row1_v, out1_v, 0)
        pltpu.make_async_copy(out1_v, enc_hbm.at[wid + _L], sem1).start()

    o0.wait()
    o2.wait()

    @pl.when(wid < ns - _L)
    def _drain1():
        pltpu.make_async_copy(out1_v, enc_hbm.at[wid + _L], sem1).wait()


def _tc_body(x_ref, enc_ref, o_ref, a_ref):
    @pl.when(pl.program_id(0) == 0)
    def _build_add():
        ns = a_ref.shape[0]
        ale = enc_ref[:ns, :]
        ape = enc_ref[ns:, :]
        a_ref[...] = ale[:, None, :] + ape[None, :, :]

    o_ref[...] = x_ref[...] + a_ref[...][None]


def kernel(x, learned_table, person_table, num_people):
    del num_people
    b, s, p, d = x.shape
    pe_rows = lax.slice(person_table, (0, 0), (p, person_table.shape[1]))

    sc_call = functools.partial(
        pl.kernel,
        mesh=plsc.VectorSubcoreMesh(core_axis_name="c", subcore_axis_name="s", num_cores=1),
        out_type=jax.ShapeDtypeStruct((s + p, d), jnp.float32),
        scratch_types=[
            pltpu.VMEM((d // 2,), jnp.float32),
            pltpu.VMEM((d // 2,), jnp.float32),
            pltpu.VMEM((d // 2,), jnp.float32),
            pltpu.VMEM((d,), jnp.float32),
            pltpu.VMEM((d,), jnp.float32),
            pltpu.VMEM((d,), jnp.float32),
            pltpu.SemaphoreType.DMA,
            pltpu.SemaphoreType.DMA,
            pltpu.SemaphoreType.DMA,
        ],
        compiler_params=pltpu.CompilerParams(needs_layout_passes=False),
    )(_sc_lookup)
    enc = sc_call(learned_table, pe_rows)

    grid = (b // B_BLK,)
    return pl.pallas_call(
        _tc_body,
        grid=grid,
        in_specs=[
            pl.BlockSpec((B_BLK, s, p, d), lambda i: (i, 0, 0, 0)),
            pl.BlockSpec((s + p, d), lambda i: (0, 0)),
        ],
        out_specs=pl.BlockSpec((B_BLK, s, p, d), lambda i: (i, 0, 0, 0)),
        out_shape=jax.ShapeDtypeStruct(x.shape, x.dtype),
        scratch_shapes=[pltpu.VMEM((s, p, d), jnp.float32)],
        compiler_params=pltpu.CompilerParams(
            dimension_semantics=("arbitrary",),
        ),
    )(x, enc)


# speedup vs baseline: 1.0632x; 1.0012x over previous
import functools

import jax
import jax.numpy as jnp
from jax import lax
from jax.experimental import pallas as pl
from jax.experimental.pallas import tpu as pltpu
from jax.experimental.pallas import tpu_sc as plsc

B_BLK = 32
_L = 16


def _vec_sqrt(x):
    y = 0.5 * (x + 1.0)
    for _ in range(14):
        y = 0.5 * (y + x / y)
    return y


def _sc_compute_row(row_v, out_v, parity):
    d = 128
    nch = d // _L
    ss = jnp.zeros((_L,), jnp.float32)
    for c in range(nch):
        v = row_v[pl.ds(c * _L, _L)]
        ss = ss + v * v
    tot_s = ss[0]
    for i in range(1, _L):
        tot_s = tot_s + ss[i]
    tot = jnp.full((_L,), tot_s, jnp.float32)
    norm = _vec_sqrt(tot)
    scale = jnp.where(norm > 1.0, 1.0 / (norm + 1e-7), jnp.float32(1.0))
    zero = jnp.zeros((_L,), jnp.float32)
    for j in range(2 * nch):
        out_v[pl.ds(j * _L, _L)] = zero
    base_iota = lax.iota(jnp.int32, _L)
    for c in range(nch):
        v = row_v[pl.ds(c * _L, _L)] * scale
        idx = 2 * (c * _L + base_iota) + parity
        plsc.store_scatter(out_v, [idx], v)


def _sc_lookup(le_hbm, pe_hbm, enc_hbm, row0_v, row1_v, row2_v, out0_v, out1_v, out2_v, sem0, sem1, sem2):
    wid = lax.axis_index("s")
    ns = le_hbm.shape[0]

    cp0 = pltpu.make_async_copy(le_hbm.at[wid], row0_v, sem0)
    cp2 = pltpu.make_async_copy(pe_hbm.at[wid], row2_v, sem2)
    cp0.start()
    cp2.start()

    @pl.when(wid < ns - _L)
    def _start1():
        pltpu.make_async_copy(le_hbm.at[wid + _L], row1_v, sem1).start()

    cp0.wait()
    _sc_compute_row(row0_v, out0_v, 0)
    o0 = pltpu.make_async_copy(out0_v, enc_hbm.at[wid], sem0)
    o0.start()

    cp2.wait()
    _sc_compute_row(row2_v, out2_v, 1)
    o2 = pltpu.make_async_copy(out2_v, enc_hbm.at[ns + wid], sem2)
    o2.start()

    @pl.when(wid < ns - _L)
    def _job1():
        pltpu.make_async_copy(le_hbm.at[wid + _L], row1_v, sem1).wait()
        _sc_compute_row(row1_v, out1_v, 0)
        pltpu.make_async_copy(out1_v, enc_hbm.at[wid + _L], sem1).start()

    o0.wait()
    o2.wait()

    @pl.when(wid < ns - _L)
    def _drain1():
        pltpu.make_async_copy(out1_v, enc_hbm.at[wid + _L], sem1).wait()


def _tc_body(x_ref, enc_ref, o_ref, a_ref):
    @pl.when(pl.program_id(0) == 0)
    def _build_add():
        ns = a_ref.shape[0]
        ale = enc_ref[:ns, :]
        ape = enc_ref[ns:, :]
        a_ref[...] = ale[:, None, :] + ape[None, :, :]

    o_ref[...] = x_ref[...] + a_ref[...][None]


def kernel(x, learned_table, person_table, num_people):
    del num_people
    b, s, p, d = x.shape
    pe_rows = lax.slice(person_table, (0, 0), (p, person_table.shape[1]))

    sc_call = functools.partial(
        pl.kernel,
        mesh=plsc.VectorSubcoreMesh(core_axis_name="c", subcore_axis_name="s", num_cores=1),
        out_type=jax.ShapeDtypeStruct((s + p, d), jnp.float32),
        scratch_types=[
            pltpu.VMEM((d // 2,), jnp.float32),
            pltpu.VMEM((d // 2,), jnp.float32),
            pltpu.VMEM((d // 2,), jnp.float32),
            pltpu.VMEM((d,), jnp.float32),
            pltpu.VMEM((d,), jnp.float32),
            pltpu.VMEM((d,), jnp.float32),
            pltpu.SemaphoreType.DMA,
            pltpu.SemaphoreType.DMA,
            pltpu.SemaphoreType.DMA,
        ],
        compiler_params=pltpu.CompilerParams(needs_layout_passes=False),
    )(_sc_lookup)
    enc = sc_call(learned_table, pe_rows)

    grid = (b // B_BLK,)
    return pl.pallas_call(
        _tc_body,
        grid=grid,
        in_specs=[
            pl.BlockSpec((B_BLK, s, p, d), lambda i: (i, 0, 0, 0)),
            pl.BlockSpec((s + p, d), lambda i: (0, 0)),
        ],
        out_specs=pl.BlockSpec((B_BLK, s, p, d), lambda i: (i, 0, 0, 0)),
        out_shape=jax.ShapeDtypeStruct(x.shape, x.dtype),
        scratch_shapes=[pltpu.VMEM((s, p, d), jnp.float32)],
        compiler_params=pltpu.CompilerParams(
            dimension_semantics=("arbitrary",),
        ),
    )(x, enc)
